# baseline (speedup 1.0000x reference)
import jax
import jax.numpy as jnp
from jax.experimental import pallas as pl
from jax.experimental.pallas import tpu as pltpu

_BLOCK_ROWS = 1024


def _copy_body(img_in, txt_in, img_out, txt_out):
    img_out[...] = img_in[...]
    txt_out[...] = txt_in[...]


def kernel(image_feat, text_feat):
    n_rows, n_cols = image_feat.shape
    grid = (n_rows // _BLOCK_ROWS,)
    spec = pl.BlockSpec((_BLOCK_ROWS, n_cols), lambda i: (i, 0))
    out = pl.pallas_call(
        _copy_body,
        grid=grid,
        in_specs=[spec, spec],
        out_specs=[spec, spec],
        out_shape=[
            jax.ShapeDtypeStruct(image_feat.shape, image_feat.dtype),
            jax.ShapeDtypeStruct(text_feat.shape, text_feat.dtype),
        ],
        compiler_params=pltpu.CompilerParams(
            dimension_semantics=("parallel",),
        ),
    )(image_feat, text_feat)
    return (out[0], out[1])
